# baseline (speedup 1.0000x reference)
import functools
import jax
import jax.numpy as jnp
from jax import lax
from jax.experimental import pallas as pl
from jax.experimental.pallas import tpu as pltpu
from jax.experimental.pallas import tpu_sc as plsc

DIM = 1024
SEG = 4
SEG_DIM = DIM // SEG
E = 8
TOPK = 2
HID = 4 * DIM
EHID = 4 * SEG_DIM
NSH = 1
T = 2048

TT = 256
NTT = T // TT
KH = 1024
NKH = HID // KH

N = SEG * T
PAIRS = N * TOPK
TILE = 256
NT = PAIRS // TILE + E
NP = NT * TILE

NC = 2
NS = 16
NW = NC * NS
PPW = PAIRS // NW
RPW = NP // NW
GCH = 96
MROWS = N // NW
GC = 32


def _router_body(x_ref, wr_ref, br_ref, tw_ref, ti_ref, rk_ref, cnt_ref,
                 aux_ref, acc_ref):
    tt = pl.program_id(0)

    @pl.when(tt == 0)
    def _():
        acc_ref[...] = jnp.zeros((2, E), jnp.float32)

    carry = acc_ref[0:1, :]
    iota8 = lax.broadcasted_iota(jnp.int32, (TT, E), 1)
    w_cols = []
    i_cols = []
    for s in range(SEG):
        x = x_ref[:, s * SEG_DIM:(s + 1) * SEG_DIM]
        logits = jnp.dot(x, wr_ref[...], preferred_element_type=jnp.float32)
        logits = logits + br_ref[...]
        m1 = jnp.max(logits, axis=1, keepdims=True)
        p = jnp.exp(logits - m1)
        p = p / jnp.sum(p, axis=1, keepdims=True)
        p1 = jnp.max(p, axis=1, keepdims=True)
        i1 = jnp.min(jnp.where(p == p1, iota8, E), axis=1, keepdims=True)
        p_m = jnp.where(iota8 == i1, -jnp.inf, p)
        p2 = jnp.max(p_m, axis=1, keepdims=True)
        i2 = jnp.min(jnp.where(p_m == p2, iota8, E), axis=1, keepdims=True)
        w_cols += [p1, p2]
        i_cols += [i1, i2]
    tw_t = jnp.concatenate(w_cols, axis=1)
    ti_t = jnp.concatenate(i_cols, axis=1)
    tw_ref[...] = tw_t
    ti_ref[...] = ti_t

    oh_cols = [jnp.sum((ti_t == e).astype(jnp.float32), axis=1, keepdims=True)
               for e in range(E)]
    oh_cnt = jnp.concatenate(oh_cols, axis=1)
    r_io = lax.broadcasted_iota(jnp.int32, (TT, TT), 0)
    c_io = lax.broadcasted_iota(jnp.int32, (TT, TT), 1)
    ltri = (r_io > c_io).astype(jnp.float32)
    excl = jnp.dot(ltri, oh_cnt, preferred_element_type=jnp.float32)
    tot = jnp.sum(oh_cnt, axis=0, keepdims=True)
    base = carry + excl

    ranksel = jnp.zeros((TT, E), jnp.float32)
    for e in range(E):
        ranksel = jnp.where(ti_t == e, base[:, e:e + 1], ranksel)
    wrc_cols = [jnp.zeros((TT, 1), jnp.float32)]
    for c in range(1, E):
        acc_c = jnp.zeros((TT, 1), jnp.float32)
        for c2 in range(c):
            acc_c = acc_c + (ti_t[:, c2:c2 + 1] == ti_t[:, c:c + 1]
                             ).astype(jnp.float32)
        wrc_cols.append(acc_c)
    wrc = jnp.concatenate(wrc_cols, axis=1)
    rk_ref[...] = (ranksel + wrc).astype(jnp.int32)

    sw_cols = []
    for e in range(E):
        sw_e = jnp.sum(jnp.where(ti_t == e, tw_t, 0.0), axis=0, keepdims=True)
        sw_cols.append(jnp.sum(sw_e, axis=1, keepdims=True))
    sw = jnp.concatenate(sw_cols, axis=1)
    acc_ref[...] = acc_ref[...] + jnp.concatenate([tot, sw], axis=0)

    @pl.when(tt == pl.num_programs(0) - 1)
    def _():
        a = acc_ref[...]
        cnt_ref[...] = a[0:1, :]
        n_tok = jnp.float32(N)
        aux_ref[...] = (jnp.float32(E) *
                        jnp.sum((a[0:1, :] / n_tok) * (a[1:2, :] / n_tok),
                                keepdims=True).reshape(1, 1))


def _shared_body(x_ref, w1_ref, b1_ref, w2_ref, b2_ref, out_ref):
    k = pl.program_id(1)
    h = jnp.dot(x_ref[...], w1_ref[...], preferred_element_type=jnp.float32)
    h = jnp.maximum(h + b1_ref[...], 0.0)
    contrib = jnp.dot(h, w2_ref[...], preferred_element_type=jnp.float32)

    @pl.when(k == 0)
    def _():
        out_ref[...] = contrib + b2_ref[...]

    @pl.when(k > 0)
    def _():
        out_ref[...] = out_ref[...] + contrib


def _group_body(eid_ref, xs_ref, w_ref, we1_ref, be1_ref, we2_ref, be2_ref,
                out_ref):
    x = xs_ref[...]
    h = jnp.dot(x, we1_ref[0], preferred_element_type=jnp.float32)
    h = jnp.maximum(h + be1_ref[0], 0.0)
    y = jnp.dot(h, we2_ref[0], preferred_element_type=jnp.float32)
    y = y + be2_ref[0]
    out_ref[...] = y * w_ref[...]


PROW = PPW // 128
PPC = PAIRS // NS
PRC = PPC // 128
ZPC = NP // NS


def _dispatch_body(ti_hbm, rk_hbm, tw_hbm, off_hbm, xf_hbm,
                   pos_hbm, w_hbm, xs_hbm,
                   off_v, ti_v, rk_v, tw_v, pos_v, tok_v,
                   zi_v, zf_v, idx_v, wr_v, xr0_v, xr1_v,
                   spm_tok, spm_w, sem, semg0, semg1, semw0, semw1):
    cid = lax.axis_index("c")
    sid = lax.axis_index("s")
    wid = sid * NC + cid

    for j in range(ZPC // 16):
        zi_v[pl.ds(j * 16, 16)] = jnp.zeros((16,), jnp.int32)
        zf_v[pl.ds(j * 16, 16)] = jnp.zeros((16,), jnp.float32)
    z0 = pltpu.async_copy(zi_v, spm_tok.at[pl.ds(sid * ZPC, ZPC)], semg0)
    z1 = pltpu.async_copy(zf_v, spm_w.at[pl.ds(sid * ZPC, ZPC)], semg1)
    pbase = sid * PPC
    pltpu.sync_copy(ti_hbm.at[pl.ds(pbase, PPC)], ti_v)
    pltpu.sync_copy(rk_hbm.at[pl.ds(pbase, PPC)], rk_v)
    pltpu.sync_copy(tw_hbm.at[pl.ds(sid * PRC, PRC)], tw_v)
    pltpu.sync_copy(off_hbm, off_v)
    z0.wait()
    z1.wait()
    plsc.subcore_barrier()

    lane = lax.iota(jnp.int32, 16)
    offr = [off_v[e] for e in range(E)]
    for j in range(PPC // 16):
        b = j * 16
        e16 = ti_v[pl.ds(b, 16)]
        off16 = jnp.zeros((16,), jnp.int32)
        for e in range(E):
            off16 = jnp.where(e16 == e, offr[e], off16)
        pos16 = off16 + rk_v[pl.ds(b, 16)]
        pos_v[b // 128, pl.ds(b % 128, 16)] = pos16
        p16 = pbase + b + lane
        t16 = p16 >> 3
        s16 = (p16 & 7) >> 1
        tok_v[b // 128, pl.ds(b % 128, 16)] = t16 * SEG + s16

    scats = []
    for j in range(PRC):
        scats.append(pltpu.async_copy(
            tok_v.at[j], spm_tok.at[pos_v.at[j]], sem, add=True))
        scats.append(pltpu.async_copy(
            tw_v.at[j], spm_w.at[pos_v.at[j]], sem, add=True))

    @pl.when(cid == 0)
    def _():
        pltpu.sync_copy(pos_v, pos_hbm.at[pl.ds(sid * PRC, PRC)])

    for s in scats:
        s.wait()
    plsc.subcore_barrier()

    rbase = wid * RPW
    pltpu.sync_copy(spm_tok.at[pl.ds(rbase, RPW)], idx_v)
    pltpu.sync_copy(spm_w.at[pl.ds(rbase, RPW)], wr_v)
    pltpu.sync_copy(wr_v, w_hbm.at[pl.ds(rbase, RPW)])
    NG = RPW // GCH
    bufs = [xr0_v, xr1_v]
    gsems = [semg0, semg1]
    wsems = [semw0, semw1]
    gd = [None, None]
    wd = [None, None]

    def fire(j):
        bj = j % 2
        gd[bj] = pltpu.async_copy(
            xf_hbm.at[idx_v.at[pl.ds(j * GCH, GCH)]], bufs[bj], gsems[bj])

    fire(0)
    for j in range(NG):
        bj = j % 2
        nb = (j + 1) % 2
        if j + 1 < NG:
            if wd[nb] is not None:
                wd[nb].wait()
            fire(j + 1)
        gd[bj].wait()
        wd[bj] = pltpu.async_copy(
            bufs[bj], xs_hbm.at[pl.ds(rbase + j * GCH, GCH)], wsems[bj])
    for d in wd:
        if d is not None:
            d.wait()


def _combine_body(pos_hbm, ys_hbm, sh_hbm, out_hbm,
                  posl_v, y0_v, y1_v, s0_v, s1_v, o0_v, o1_v,
                  sy0, sy1, ss0, ss1, sw0, sw1):
    wid = lax.axis_index("s") * NC + lax.axis_index("c")
    mbase = wid * MROWS
    pltpu.sync_copy(pos_hbm.at[pl.ds(wid * PROW, PROW)], posl_v)
    NSUB = MROWS // GC
    ybufs = [y0_v, y1_v]
    sbufs = [s0_v, s1_v]
    obufs = [o0_v, o1_v]
    ysems = [sy0, sy1]
    ssems = [ss0, ss1]
    wsems = [sw0, sw1]
    gy = [None, None]
    gs = [None, None]
    wd = [None, None]

    def fire(j):
        bj = j % 2
        idx = posl_v.at[j // 2, pl.ds((j % 2) * (TOPK * GC), TOPK * GC)]
        gy[bj] = pltpu.async_copy(ys_hbm.at[idx], ybufs[bj], ysems[bj])
        gs[bj] = pltpu.async_copy(sh_hbm.at[pl.ds(mbase + j * GC, GC)],
                                  sbufs[bj], ssems[bj])

    fire(0)
    for j in range(NSUB):
        bj = j % 2
        nb = (j + 1) % 2
        if j + 1 < NSUB:
            if wd[nb] is not None:
                wd[nb].wait()
            fire(j + 1)
        gy[bj].wait()
        gs[bj].wait()
        y_v = ybufs[bj]
        sh_v = sbufs[bj]
        out_v = obufs[bj]

        def row_fn(r, _):
            for k in range(SEG_DIM // 16):
                a = y_v[2 * r, pl.ds(k * 16, 16)]
                b = y_v[2 * r + 1, pl.ds(k * 16, 16)]
                c = sh_v[r, pl.ds(k * 16, 16)]
                out_v[r, pl.ds(k * 16, 16)] = a + b + c
            return 0

        lax.fori_loop(0, GC, row_fn, 0)
        wd[bj] = pltpu.async_copy(
            out_v, out_hbm.at[pl.ds(mbase + j * GC, GC)], wsems[bj])
    for d in wd:
        if d is not None:
            d.wait()


@functools.cache
def _sc_kernels():
    sc_mesh = plsc.VectorSubcoreMesh(core_axis_name="c", subcore_axis_name="s")
    dispatch = _make_dispatch(sc_mesh)
    combine = _make_combine(sc_mesh)
    return dispatch, combine


def _make_dispatch(_sc_mesh):
    return functools.partial(
    pl.kernel,
    out_type=[
        jax.ShapeDtypeStruct((PAIRS // 128, 128), jnp.int32),
        jax.ShapeDtypeStruct((NP,), jnp.float32),
        jax.ShapeDtypeStruct((NP, SEG_DIM), jnp.float32),
    ],
    mesh=_sc_mesh,
    scratch_types=[
        pltpu.VMEM((E, 16), jnp.int32),
        pltpu.VMEM((PPC,), jnp.int32),
        pltpu.VMEM((PPC,), jnp.int32),
        pltpu.VMEM((PRC, 128), jnp.float32),
        pltpu.VMEM((PRC, 128), jnp.int32),
        pltpu.VMEM((PRC, 128), jnp.int32),
        pltpu.VMEM((ZPC,), jnp.int32),
        pltpu.VMEM((ZPC,), jnp.float32),
        pltpu.VMEM((RPW,), jnp.int32),
        pltpu.VMEM((RPW,), jnp.float32),
        pltpu.VMEM((GCH, SEG_DIM), jnp.float32),
        pltpu.VMEM((GCH, SEG_DIM), jnp.float32),
        pltpu.VMEM_SHARED((NP,), jnp.int32),
        pltpu.VMEM_SHARED((NP,), jnp.float32),
        pltpu.SemaphoreType.DMA,
        pltpu.SemaphoreType.DMA,
        pltpu.SemaphoreType.DMA,
        pltpu.SemaphoreType.DMA,
        pltpu.SemaphoreType.DMA,
    ],
)(_dispatch_body)


def _make_combine(_sc_mesh):
    return functools.partial(
    pl.kernel,
    out_type=jax.ShapeDtypeStruct((N, SEG_DIM), jnp.float32),
    mesh=_sc_mesh,
    scratch_types=[
        pltpu.VMEM((PROW, 128), jnp.int32),
        pltpu.VMEM((TOPK * GC, SEG_DIM), jnp.float32),
        pltpu.VMEM((TOPK * GC, SEG_DIM), jnp.float32),
        pltpu.VMEM((GC, SEG_DIM), jnp.float32),
        pltpu.VMEM((GC, SEG_DIM), jnp.float32),
        pltpu.VMEM((GC, SEG_DIM), jnp.float32),
        pltpu.VMEM((GC, SEG_DIM), jnp.float32),
        pltpu.SemaphoreType.DMA,
        pltpu.SemaphoreType.DMA,
        pltpu.SemaphoreType.DMA,
        pltpu.SemaphoreType.DMA,
        pltpu.SemaphoreType.DMA,
        pltpu.SemaphoreType.DMA,
    ],
)(_combine_body)


@jax.jit
def kernel(X, Ws1, bs1, Ws2, bs2, Wr, br, We1, be1, We2, be2):
    b, t, c = X.shape
    X2 = X.reshape(t, c)
    br2 = br.reshape(1, E)

    top_w, top_idx, rank, counts, aux = pl.pallas_call(
        _router_body,
        grid=(NTT,),
        in_specs=[
            pl.BlockSpec((TT, DIM), lambda tt: (tt, 0)),
            pl.BlockSpec((SEG_DIM, E), lambda tt: (0, 0)),
            pl.BlockSpec((1, E), lambda tt: (0, 0)),
        ],
        out_specs=[
            pl.BlockSpec((TT, SEG * TOPK), lambda tt: (tt, 0)),
            pl.BlockSpec((TT, SEG * TOPK), lambda tt: (tt, 0)),
            pl.BlockSpec((TT, SEG * TOPK), lambda tt: (tt, 0)),
            pl.BlockSpec((1, E), lambda tt: (0, 0)),
            pl.BlockSpec((1, 1), lambda tt: (0, 0)),
        ],
        out_shape=[
            jax.ShapeDtypeStruct((T, SEG * TOPK), jnp.float32),
            jax.ShapeDtypeStruct((T, SEG * TOPK), jnp.int32),
            jax.ShapeDtypeStruct((T, SEG * TOPK), jnp.int32),
            jax.ShapeDtypeStruct((1, E), jnp.float32),
            jax.ShapeDtypeStruct((1, 1), jnp.float32),
        ],
        scratch_shapes=[pltpu.VMEM((2, E), jnp.float32)],
    )(X2, Wr, br2)

    counts_i = jnp.round(counts.reshape(E)).astype(jnp.int32)
    pc = ((counts_i + TILE - 1) // TILE) * TILE
    ends = jnp.cumsum(pc)
    offsets = (ends - pc).astype(jnp.int32)
    tile_end = (ends // TILE).astype(jnp.int32)
    eid = jnp.minimum(
        jnp.sum((jnp.arange(NT)[:, None] >= tile_end[None, :]).astype(
            jnp.int32), axis=1), E - 1).astype(jnp.int32)

    off_b = jnp.broadcast_to(offsets[:, None], (E, 16))
    _dispatch, _combine = _sc_kernels()
    pos, w_sorted, xs = _dispatch(
        top_idx.reshape(PAIRS), rank.reshape(PAIRS),
        top_w.reshape(PAIRS // 128, 128), off_b, X2.reshape(N, SEG_DIM))

    shared = pl.pallas_call(
        _shared_body,
        grid=(NTT, NKH),
        in_specs=[
            pl.BlockSpec((TT, DIM), lambda tt, k: (tt, 0)),
            pl.BlockSpec((DIM, KH), lambda tt, k: (0, k)),
            pl.BlockSpec((1, KH), lambda tt, k: (0, k)),
            pl.BlockSpec((KH, DIM), lambda tt, k: (k, 0)),
            pl.BlockSpec((1, DIM), lambda tt, k: (0, 0)),
        ],
        out_specs=pl.BlockSpec((TT, DIM), lambda tt, k: (tt, 0)),
        out_shape=jax.ShapeDtypeStruct((T, DIM), jnp.float32),
    )(X2, Ws1[0], bs1.reshape(NSH, 1, HID)[0], Ws2[0],
      bs2.reshape(NSH, 1, DIM)[0])

    ys = pl.pallas_call(
        _group_body,
        grid_spec=pltpu.PrefetchScalarGridSpec(
            num_scalar_prefetch=1,
            grid=(NT,),
            in_specs=[
                pl.BlockSpec((TILE, SEG_DIM), lambda g, e: (g, 0)),
                pl.BlockSpec((TILE, 1), lambda g, e: (g, 0)),
                pl.BlockSpec((1, SEG_DIM, EHID), lambda g, e: (e[g], 0, 0)),
                pl.BlockSpec((1, 1, EHID), lambda g, e: (e[g], 0, 0)),
                pl.BlockSpec((1, EHID, SEG_DIM), lambda g, e: (e[g], 0, 0)),
                pl.BlockSpec((1, 1, SEG_DIM), lambda g, e: (e[g], 0, 0)),
            ],
            out_specs=pl.BlockSpec((TILE, SEG_DIM), lambda g, e: (g, 0)),
        ),
        out_shape=jax.ShapeDtypeStruct((NP, SEG_DIM), jnp.float32),
    )(eid, xs, w_sorted.reshape(NP, 1), We1, be1.reshape(E, 1, EHID), We2,
      be2.reshape(E, 1, SEG_DIM))

    out_flat = _combine(pos, ys, shared.reshape(N, SEG_DIM))

    return (out_flat.reshape(b, t, c), aux[0, 0])


# speedup vs baseline: 1.0045x; 1.0045x over previous
import functools
import jax
import jax.numpy as jnp
from jax import lax
from jax.experimental import pallas as pl
from jax.experimental.pallas import tpu as pltpu
from jax.experimental.pallas import tpu_sc as plsc

DIM = 1024
SEG = 4
SEG_DIM = DIM // SEG
E = 8
TOPK = 2
HID = 4 * DIM
EHID = 4 * SEG_DIM
NSH = 1
T = 2048

TT = 256
NTT = T // TT
KH = 1024
NKH = HID // KH

N = SEG * T
PAIRS = N * TOPK
TILE = 256
NT = PAIRS // TILE + E
NP = NT * TILE

NC = 2
NS = 16
NW = NC * NS
PPW = PAIRS // NW
RPW = NP // NW
GCH = 96
MROWS = N // NW
GC = 32


def _router_body(x_ref, wr_ref, br_ref, tw_ref, ti_ref, rk_ref, cnt_ref,
                 aux_ref, acc_ref):
    tt = pl.program_id(0)

    @pl.when(tt == 0)
    def _():
        acc_ref[...] = jnp.zeros((2, E), jnp.float32)

    carry = acc_ref[0:1, :]
    iota8 = lax.broadcasted_iota(jnp.int32, (TT, E), 1)
    w_cols = []
    i_cols = []
    for s in range(SEG):
        x = x_ref[:, s * SEG_DIM:(s + 1) * SEG_DIM]
        logits = jnp.dot(x, wr_ref[...], preferred_element_type=jnp.float32)
        logits = logits + br_ref[...]
        m1 = jnp.max(logits, axis=1, keepdims=True)
        p = jnp.exp(logits - m1)
        p = p / jnp.sum(p, axis=1, keepdims=True)
        p1 = jnp.max(p, axis=1, keepdims=True)
        i1 = jnp.min(jnp.where(p == p1, iota8, E), axis=1, keepdims=True)
        p_m = jnp.where(iota8 == i1, -jnp.inf, p)
        p2 = jnp.max(p_m, axis=1, keepdims=True)
        i2 = jnp.min(jnp.where(p_m == p2, iota8, E), axis=1, keepdims=True)
        w_cols += [p1, p2]
        i_cols += [i1, i2]
    tw_t = jnp.concatenate(w_cols, axis=1)
    ti_t = jnp.concatenate(i_cols, axis=1)
    tw_ref[...] = tw_t
    ti_ref[...] = ti_t

    oh_cols = [jnp.sum((ti_t == e).astype(jnp.float32), axis=1, keepdims=True)
               for e in range(E)]
    oh_cnt = jnp.concatenate(oh_cols, axis=1)
    r_io = lax.broadcasted_iota(jnp.int32, (TT, TT), 0)
    c_io = lax.broadcasted_iota(jnp.int32, (TT, TT), 1)
    ltri = (r_io > c_io).astype(jnp.float32)
    excl = jnp.dot(ltri, oh_cnt, preferred_element_type=jnp.float32)
    tot = jnp.sum(oh_cnt, axis=0, keepdims=True)
    base = carry + excl

    ranksel = jnp.zeros((TT, E), jnp.float32)
    for e in range(E):
        ranksel = jnp.where(ti_t == e, base[:, e:e + 1], ranksel)
    wrc_cols = [jnp.zeros((TT, 1), jnp.float32)]
    for c in range(1, E):
        acc_c = jnp.zeros((TT, 1), jnp.float32)
        for c2 in range(c):
            acc_c = acc_c + (ti_t[:, c2:c2 + 1] == ti_t[:, c:c + 1]
                             ).astype(jnp.float32)
        wrc_cols.append(acc_c)
    wrc = jnp.concatenate(wrc_cols, axis=1)
    rk_ref[...] = (ranksel + wrc).astype(jnp.int32)

    sw_cols = []
    for e in range(E):
        sw_e = jnp.sum(jnp.where(ti_t == e, tw_t, 0.0), axis=0, keepdims=True)
        sw_cols.append(jnp.sum(sw_e, axis=1, keepdims=True))
    sw = jnp.concatenate(sw_cols, axis=1)
    acc_ref[...] = acc_ref[...] + jnp.concatenate([tot, sw], axis=0)

    @pl.when(tt == pl.num_programs(0) - 1)
    def _():
        a = acc_ref[...]
        cnt_ref[...] = a[0:1, :]
        n_tok = jnp.float32(N)
        aux_ref[...] = (jnp.float32(E) *
                        jnp.sum((a[0:1, :] / n_tok) * (a[1:2, :] / n_tok),
                                keepdims=True).reshape(1, 1))


def _shared_body(x_ref, w1_ref, b1_ref, w2_ref, b2_ref, out_ref):
    k = pl.program_id(1)
    h = jnp.dot(x_ref[...], w1_ref[...], preferred_element_type=jnp.float32)
    h = jnp.maximum(h + b1_ref[...], 0.0)
    contrib = jnp.dot(h, w2_ref[...], preferred_element_type=jnp.float32)

    @pl.when(k == 0)
    def _():
        out_ref[...] = contrib + b2_ref[...]

    @pl.when(k > 0)
    def _():
        out_ref[...] = out_ref[...] + contrib


def _group_body(eid_ref, xs_ref, we1_ref, be1_ref, we2_ref, be2_ref,
                out_ref):
    x = xs_ref[...]
    h = jnp.dot(x, we1_ref[0], preferred_element_type=jnp.float32)
    h = jnp.maximum(h + be1_ref[0], 0.0)
    y = jnp.dot(h, we2_ref[0], preferred_element_type=jnp.float32)
    out_ref[...] = y + be2_ref[0]


PROW = PPW // 128
PPC = PAIRS // NS
PRC = PPC // 128
ZPC = NP // NS


def _dispatch_body(ti_hbm, rk_hbm, off_hbm, xf_hbm,
                   pos_hbm, xs_hbm,
                   off_v, ti_v, rk_v, pos_v, pair_v,
                   zi_v, idx_v, xr0_v, xr1_v,
                   spm_pair, sem, semg0, semg1, semw0, semw1):
    cid = lax.axis_index("c")
    sid = lax.axis_index("s")
    wid = sid * NC + cid

    for j in range(ZPC // 16):
        zi_v[pl.ds(j * 16, 16)] = jnp.zeros((16,), jnp.int32)
    z0 = pltpu.async_copy(zi_v, spm_pair.at[pl.ds(sid * ZPC, ZPC)], semg0)
    pbase = sid * PPC
    pltpu.sync_copy(ti_hbm.at[pl.ds(pbase, PPC)], ti_v)
    pltpu.sync_copy(rk_hbm.at[pl.ds(pbase, PPC)], rk_v)
    pltpu.sync_copy(off_hbm, off_v)
    z0.wait()
    plsc.subcore_barrier()

    lane = lax.iota(jnp.int32, 16)
    offr = [off_v[e] for e in range(E)]
    for j in range(PPC // 16):
        b = j * 16
        e16 = ti_v[pl.ds(b, 16)]
        off16 = jnp.zeros((16,), jnp.int32)
        for e in range(E):
            off16 = jnp.where(e16 == e, offr[e], off16)
        pos_v[b // 128, pl.ds(b % 128, 16)] = off16 + rk_v[pl.ds(b, 16)]
        pair_v[b // 128, pl.ds(b % 128, 16)] = pbase + b + lane

    scats = []
    for j in range(PRC):
        scats.append(pltpu.async_copy(
            pair_v.at[j], spm_pair.at[pos_v.at[j]], sem, add=True))

    @pl.when(cid == 0)
    def _():
        pltpu.sync_copy(pos_v, pos_hbm.at[pl.ds(sid * PRC, PRC)])

    for s in scats:
        s.wait()
    plsc.subcore_barrier()

    rbase = wid * RPW
    pltpu.sync_copy(spm_pair.at[pl.ds(rbase, RPW)], idx_v)
    for j in range(RPW // 16):
        p16 = idx_v[pl.ds(j * 16, 16)]
        idx_v[pl.ds(j * 16, 16)] = (p16 >> 3) * SEG + ((p16 & 7) >> 1)
    NG = RPW // GCH
    bufs = [xr0_v, xr1_v]
    gsems = [semg0, semg1]
    wsems = [semw0, semw1]
    gd = [None, None]
    wd = [None, None]

    def fire(j):
        bj = j % 2
        gd[bj] = pltpu.async_copy(
            xf_hbm.at[idx_v.at[pl.ds(j * GCH, GCH)]], bufs[bj], gsems[bj])

    fire(0)
    for j in range(NG):
        bj = j % 2
        nb = (j + 1) % 2
        if j + 1 < NG:
            if wd[nb] is not None:
                wd[nb].wait()
            fire(j + 1)
        gd[bj].wait()
        wd[bj] = pltpu.async_copy(
            bufs[bj], xs_hbm.at[pl.ds(rbase + j * GCH, GCH)], wsems[bj])
    for d in wd:
        if d is not None:
            d.wait()


def _combine_body(pos_hbm, ys_hbm, sh_hbm, wb_hbm, out_hbm,
                  posl_v, y0_v, y1_v, s0_v, s1_v, b0_v, b1_v, o0_v, o1_v,
                  sy0, sy1, ss0, ss1, sb0, sb1, sw0, sw1):
    wid = lax.axis_index("s") * NC + lax.axis_index("c")
    mbase = wid * MROWS
    pltpu.sync_copy(pos_hbm.at[pl.ds(wid * PROW, PROW)], posl_v)
    NSUB = MROWS // GC
    ybufs = [y0_v, y1_v]
    sbufs = [s0_v, s1_v]
    bbufs = [b0_v, b1_v]
    obufs = [o0_v, o1_v]
    ysems = [sy0, sy1]
    ssems = [ss0, ss1]
    bsems = [sb0, sb1]
    wsems = [sw0, sw1]
    gy = [None, None]
    gs = [None, None]
    gb = [None, None]
    wd = [None, None]

    def fire(j):
        bj = j % 2
        idx = posl_v.at[j // 2, pl.ds((j % 2) * (TOPK * GC), TOPK * GC)]
        gy[bj] = pltpu.async_copy(ys_hbm.at[idx], ybufs[bj], ysems[bj])
        gs[bj] = pltpu.async_copy(sh_hbm.at[pl.ds(mbase + j * GC, GC)],
                                  sbufs[bj], ssems[bj])
        gb[bj] = pltpu.async_copy(wb_hbm.at[pl.ds(mbase + j * GC, GC)],
                                  bbufs[bj], bsems[bj])

    fire(0)
    for j in range(NSUB):
        bj = j % 2
        nb = (j + 1) % 2
        if j + 1 < NSUB:
            if wd[nb] is not None:
                wd[nb].wait()
            fire(j + 1)
        gy[bj].wait()
        gs[bj].wait()
        gb[bj].wait()
        y_v = ybufs[bj]
        sh_v = sbufs[bj]
        wb_v = bbufs[bj]
        out_v = obufs[bj]

        def row_fn(r, _):
            w0 = wb_v[r, pl.ds(0, 16)]
            w1 = wb_v[r, pl.ds(16, 16)]
            for k in range(SEG_DIM // 16):
                a = y_v[2 * r, pl.ds(k * 16, 16)]
                bb = y_v[2 * r + 1, pl.ds(k * 16, 16)]
                c = sh_v[r, pl.ds(k * 16, 16)]
                out_v[r, pl.ds(k * 16, 16)] = a * w0 + bb * w1 + c
            return 0

        lax.fori_loop(0, GC, row_fn, 0)
        wd[bj] = pltpu.async_copy(
            out_v, out_hbm.at[pl.ds(mbase + j * GC, GC)], wsems[bj])
    for d in wd:
        if d is not None:
            d.wait()


@functools.cache
def _sc_kernels():
    sc_mesh = plsc.VectorSubcoreMesh(core_axis_name="c", subcore_axis_name="s")
    dispatch = _make_dispatch(sc_mesh)
    combine = _make_combine(sc_mesh)
    return dispatch, combine


def _make_dispatch(_sc_mesh):
    return functools.partial(
    pl.kernel,
    out_type=[
        jax.ShapeDtypeStruct((PAIRS // 128, 128), jnp.int32),
        jax.ShapeDtypeStruct((NP, SEG_DIM), jnp.float32),
    ],
    mesh=_sc_mesh,
    scratch_types=[
        pltpu.VMEM((E, 16), jnp.int32),
        pltpu.VMEM((PPC,), jnp.int32),
        pltpu.VMEM((PPC,), jnp.int32),
        pltpu.VMEM((PRC, 128), jnp.int32),
        pltpu.VMEM((PRC, 128), jnp.int32),
        pltpu.VMEM((ZPC,), jnp.int32),
        pltpu.VMEM((RPW,), jnp.int32),
        pltpu.VMEM((GCH, SEG_DIM), jnp.float32),
        pltpu.VMEM((GCH, SEG_DIM), jnp.float32),
        pltpu.VMEM_SHARED((NP,), jnp.int32),
        pltpu.SemaphoreType.DMA,
        pltpu.SemaphoreType.DMA,
        pltpu.SemaphoreType.DMA,
        pltpu.SemaphoreType.DMA,
        pltpu.SemaphoreType.DMA,
    ],
)(_dispatch_body)


def _make_combine(_sc_mesh):
    return functools.partial(
    pl.kernel,
    out_type=jax.ShapeDtypeStruct((N, SEG_DIM), jnp.float32),
    mesh=_sc_mesh,
    scratch_types=[
        pltpu.VMEM((PROW, 128), jnp.int32),
        pltpu.VMEM((TOPK * GC, SEG_DIM), jnp.float32),
        pltpu.VMEM((TOPK * GC, SEG_DIM), jnp.float32),
        pltpu.VMEM((GC, SEG_DIM), jnp.float32),
        pltpu.VMEM((GC, SEG_DIM), jnp.float32),
        pltpu.VMEM((GC, 2 * 16), jnp.float32),
        pltpu.VMEM((GC, 2 * 16), jnp.float32),
        pltpu.VMEM((GC, SEG_DIM), jnp.float32),
        pltpu.VMEM((GC, SEG_DIM), jnp.float32),
        pltpu.SemaphoreType.DMA,
        pltpu.SemaphoreType.DMA,
        pltpu.SemaphoreType.DMA,
        pltpu.SemaphoreType.DMA,
        pltpu.SemaphoreType.DMA,
        pltpu.SemaphoreType.DMA,
        pltpu.SemaphoreType.DMA,
        pltpu.SemaphoreType.DMA,
    ],
)(_combine_body)


@jax.jit
def kernel(X, Ws1, bs1, Ws2, bs2, Wr, br, We1, be1, We2, be2):
    b, t, c = X.shape
    X2 = X.reshape(t, c)
    br2 = br.reshape(1, E)

    top_w, top_idx, rank, counts, aux = pl.pallas_call(
        _router_body,
        grid=(NTT,),
        in_specs=[
            pl.BlockSpec((TT, DIM), lambda tt: (tt, 0)),
            pl.BlockSpec((SEG_DIM, E), lambda tt: (0, 0)),
            pl.BlockSpec((1, E), lambda tt: (0, 0)),
        ],
        out_specs=[
            pl.BlockSpec((TT, SEG * TOPK), lambda tt: (tt, 0)),
            pl.BlockSpec((TT, SEG * TOPK), lambda tt: (tt, 0)),
            pl.BlockSpec((TT, SEG * TOPK), lambda tt: (tt, 0)),
            pl.BlockSpec((1, E), lambda tt: (0, 0)),
            pl.BlockSpec((1, 1), lambda tt: (0, 0)),
        ],
        out_shape=[
            jax.ShapeDtypeStruct((T, SEG * TOPK), jnp.float32),
            jax.ShapeDtypeStruct((T, SEG * TOPK), jnp.int32),
            jax.ShapeDtypeStruct((T, SEG * TOPK), jnp.int32),
            jax.ShapeDtypeStruct((1, E), jnp.float32),
            jax.ShapeDtypeStruct((1, 1), jnp.float32),
        ],
        scratch_shapes=[pltpu.VMEM((2, E), jnp.float32)],
    )(X2, Wr, br2)

    counts_i = jnp.round(counts.reshape(E)).astype(jnp.int32)
    pc = ((counts_i + TILE - 1) // TILE) * TILE
    ends = jnp.cumsum(pc)
    offsets = (ends - pc).astype(jnp.int32)
    tile_end = (ends // TILE).astype(jnp.int32)
    eid = jnp.minimum(
        jnp.sum((jnp.arange(NT)[:, None] >= tile_end[None, :]).astype(
            jnp.int32), axis=1), E - 1).astype(jnp.int32)

    shared = pl.pallas_call(
        _shared_body,
        grid=(NTT, NKH),
        in_specs=[
            pl.BlockSpec((TT, DIM), lambda tt, k: (tt, 0)),
            pl.BlockSpec((DIM, KH), lambda tt, k: (0, k)),
            pl.BlockSpec((1, KH), lambda tt, k: (0, k)),
            pl.BlockSpec((KH, DIM), lambda tt, k: (k, 0)),
            pl.BlockSpec((1, DIM), lambda tt, k: (0, 0)),
        ],
        out_specs=pl.BlockSpec((TT, DIM), lambda tt, k: (tt, 0)),
        out_shape=jax.ShapeDtypeStruct((T, DIM), jnp.float32),
    )(X2, Ws1[0], bs1.reshape(NSH, 1, HID)[0], Ws2[0],
      bs2.reshape(NSH, 1, DIM)[0])

    off_b = jnp.broadcast_to(offsets[:, None], (E, 16))
    _dispatch, _combine = _sc_kernels()
    pos, xs = _dispatch(
        top_idx.reshape(PAIRS), rank.reshape(PAIRS), off_b,
        X2.reshape(N, SEG_DIM))

    ys = pl.pallas_call(
        _group_body,
        grid_spec=pltpu.PrefetchScalarGridSpec(
            num_scalar_prefetch=1,
            grid=(NT,),
            in_specs=[
                pl.BlockSpec((TILE, SEG_DIM), lambda g, e: (g, 0)),
                pl.BlockSpec((1, SEG_DIM, EHID), lambda g, e: (e[g], 0, 0)),
                pl.BlockSpec((1, 1, EHID), lambda g, e: (e[g], 0, 0)),
                pl.BlockSpec((1, EHID, SEG_DIM), lambda g, e: (e[g], 0, 0)),
                pl.BlockSpec((1, 1, SEG_DIM), lambda g, e: (e[g], 0, 0)),
            ],
            out_specs=pl.BlockSpec((TILE, SEG_DIM), lambda g, e: (g, 0)),
        ),
        out_shape=jax.ShapeDtypeStruct((NP, SEG_DIM), jnp.float32),
    )(eid, xs, We1, be1.reshape(E, 1, EHID), We2,
      be2.reshape(E, 1, SEG_DIM))

    wb = jnp.repeat(top_w.reshape(N, TOPK), 16, axis=1)

    out_flat = _combine(pos, ys, shared.reshape(N, SEG_DIM), wb)

    return (out_flat.reshape(b, t, c), aux[0, 0])
